# baseline (speedup 1.0000x reference)
import functools

import jax
import jax.numpy as jnp
from jax import lax
from jax.experimental import pallas as pl
from jax.experimental.pallas import tpu as pltpu
from jax.experimental.pallas import tpu_sc as plsc

B = 4096
HIST = 50
D = 32
NC, NS = 2, 16
NW = NC * NS
BW = B // NW
IB = BW * HIST
L = 16
NG = BW // L

UROWS = 100000
IROWS = 1000000
UCH = UROWS // 128
UREM = UROWS - UCH * 128
ICH = IROWS // 128
IREM = IROWS - ICH * 128

_PARAMS = pltpu.CompilerParams(use_tc_tiling_on_sc=True,
                               needs_layout_passes=False)
_MESH = plsc.VectorSubcoreMesh(core_axis_name="c", subcore_axis_name="s",
                               num_cores=NC, num_subcores=NS)


def _transpose_chunk(in_v, tr_v, lanes, n_gl):
    for gl in range(n_gl):
        for h in range(8):
            q = h // 2
            dvec = lanes + (16 * (h % 2))
            tr_v[gl, pl.ds(16 * h, L)] = plsc.load_gather(
                in_v, [dvec, lanes * 0 + (4 * gl + q)])


def _retile_body(user_tt_hbm, item_tt_hbm, user_tail_hbm, item_tail_hbm,
                 user_l_hbm, item_l_hbm,
                 in0_v, in1_v, tr0_v, tr1_v,
                 isem0, isem1, osem0, osem1):
    wid = lax.axis_index("s") * NC + lax.axis_index("c")
    lanes = lax.iota(jnp.int32, L)
    ins = (in0_v, in1_v)
    trs = (tr0_v, tr1_v)
    isems = (isem0, isem1)
    osems = (osem0, osem1)

    def run(src_hbm, dst_hbm, nfull):
        def fire(t, par):
            c = wid + t * NW
            pltpu.async_copy(src_hbm.at[:, pl.ds(c * 128, 128)],
                             ins[par], isems[par])

        fire(0, 0)

        def step(t, par):
            @pl.when(t + 1 < nfull)
            def _():
                fire(t + 1, 1 - par)
            pltpu.make_async_copy(
                src_hbm.at[:, pl.ds((wid + t * NW) * 128, 128)],
                ins[par], isems[par]).wait()

            @pl.when(t >= 2)
            def _():
                c_old = wid + (t - 2) * NW
                pltpu.make_async_copy(
                    trs[par], dst_hbm.at[pl.ds(c_old * 32, 32)],
                    osems[par]).wait()
            _transpose_chunk(ins[par], trs[par], lanes, 32)
            c = wid + t * NW
            pltpu.async_copy(trs[par], dst_hbm.at[pl.ds(c * 32, 32)],
                             osems[par])

        def pair(p, carry):
            step(p * 2, 0)
            step(p * 2 + 1, 1)
            return carry

        lax.fori_loop(0, nfull // 2, pair, 0)

        def drain(par, t):
            pltpu.make_async_copy(
                trs[par], dst_hbm.at[pl.ds((wid + t * NW) * 32, 32)],
                osems[par]).wait()

        drain(0, nfull - 2)
        drain(1, nfull - 1)

    run(item_tt_hbm, item_l_hbm, ICH // NW)
    run(user_tt_hbm, user_l_hbm, UCH // NW)

    c_extra = (ICH // NW) * NW + wid

    @pl.when(c_extra < ICH)
    def _():
        pltpu.sync_copy(item_tt_hbm.at[:, pl.ds(c_extra * 128, 128)], in0_v)
        _transpose_chunk(in0_v, tr0_v, lanes, 32)
        pltpu.sync_copy(tr0_v, item_l_hbm.at[pl.ds(c_extra * 32, 32)])

    @pl.when(wid == (ICH % NW))
    def _():
        pltpu.sync_copy(item_tail_hbm, tr1_v.at[pl.ds(0, 16)])
        pltpu.sync_copy(tr1_v.at[pl.ds(0, 16)],
                        item_l_hbm.at[pl.ds(IROWS * D // 128 - 16, 16)])

    u_extra = (UCH // NW) * NW + wid

    @pl.when(u_extra < UCH)
    def _():
        pltpu.sync_copy(user_tt_hbm.at[:, pl.ds(u_extra * 128, 128)], in0_v)
        _transpose_chunk(in0_v, tr0_v, lanes, 32)
        pltpu.sync_copy(tr0_v, user_l_hbm.at[pl.ds(u_extra * 32, 32)])

    @pl.when(wid == (UCH % NW))
    def _():
        pltpu.sync_copy(user_tail_hbm, in1_v.at[pl.ds(0, 8)])
        pltpu.sync_copy(in1_v.at[pl.ds(0, 8)],
                        user_l_hbm.at[pl.ds(UROWS * D // 128 - 8, 8)])


_retile = functools.partial(
    pl.kernel,
    out_type=(
        jax.ShapeDtypeStruct((UROWS * D // 128, 128), jnp.float32),
        jax.ShapeDtypeStruct((IROWS * D // 128, 128), jnp.float32),
    ),
    mesh=_MESH,
    scratch_types=[
        pltpu.VMEM((D, 128), jnp.float32),
        pltpu.VMEM((D, 128), jnp.float32),
        pltpu.VMEM((32, 128), jnp.float32),
        pltpu.VMEM((32, 128), jnp.float32),
        pltpu.SemaphoreType.DMA,
        pltpu.SemaphoreType.DMA,
        pltpu.SemaphoreType.DMA,
        pltpu.SemaphoreType.DMA,
    ],
    compiler_params=_PARAMS,
)(_retile_body)


def _extract_transpose(lines_v, col_v, blk_v, lanes):
    for g in range(NG):
        rows = lanes + (g * L)
        cols = col_v[pl.ds(g * L, L)]
        for d in range(D):
            blk_v[d, pl.ds(g * L, L)] = plsc.load_gather(
                lines_v, [rows, cols + d])


def _gather_body(user_id_hbm, items_hbm, user_l_hbm, item_l_hbm,
                 user_out_hbm, item_out_hbm,
                 uidx_v, iidx_v, lid0_v, lid1_v, col0_v, col1_v,
                 lines0_v, lines1_v, blk0_v, blk1_v,
                 gsem0, gsem1, osem0, osem1, usem):
    wid = lax.axis_index("s") * NC + lax.axis_index("c")
    b0 = wid * BW

    pltpu.sync_copy(user_id_hbm.at[pl.ds(b0, BW)], uidx_v)
    pltpu.sync_copy(items_hbm.at[pl.ds(b0 * HIST, IB)], iidx_v)

    lanes = lax.iota(jnp.int32, L)
    lanes50 = lanes * HIST

    lids = (lid0_v, lid1_v)
    cols = (col0_v, col1_v)
    lines = (lines0_v, lines1_v)
    blks = (blk0_v, blk1_v)
    gsems = (gsem0, gsem1)
    osems = (osem0, osem1)

    def prep_fire(l, par):
        for g in range(NG):
            raw = plsc.load_gather(iidx_v, [lanes50 + (g * L * HIST + l)])
            lids[par][pl.ds(g * L, L)] = lax.shift_right_logical(raw, 2)
            cols[par][pl.ds(g * L, L)] = lax.shift_left(
                lax.bitwise_and(raw, jnp.int32(3)), 5)
        pltpu.async_copy(item_l_hbm.at[lids[par]], lines[par], gsems[par])

    for g in range(NG):
        raw = uidx_v[pl.ds(g * L, L)]
        lid0_v[pl.ds(g * L, L)] = lax.shift_right_logical(raw, 2)
        col0_v[pl.ds(g * L, L)] = lax.shift_left(
            lax.bitwise_and(raw, jnp.int32(3)), 5)
    pltpu.async_copy(user_l_hbm.at[lid0_v], lines0_v, usem)
    pltpu.make_async_copy(user_l_hbm.at[lid0_v], lines0_v, usem).wait()
    _extract_transpose(lines0_v, col0_v, blk0_v, lanes)
    pltpu.sync_copy(blk0_v, user_out_hbm.at[:, pl.ds(b0, BW)])

    prep_fire(0, 1)

    def step(l, par):
        @pl.when(l + 1 < HIST)
        def _():
            prep_fire(l + 1, 1 - par)
        pltpu.make_async_copy(
            item_l_hbm.at[lids[par]], lines[par], gsems[par]).wait()

        @pl.when(l >= 2)
        def _():
            pltpu.make_async_copy(
                blks[par], item_out_hbm.at[l - 2, :, pl.ds(b0, BW)],
                osems[par]).wait()
        _extract_transpose(lines[par], cols[par], blks[par], lanes)
        pltpu.async_copy(blks[par],
                         item_out_hbm.at[l, :, pl.ds(b0, BW)], osems[par])

    def pair(p, carry):
        l = p * 2
        step(l, 1)
        step(l + 1, 0)
        return carry

    lax.fori_loop(0, HIST // 2, pair, 0)

    pltpu.make_async_copy(
        blk1_v, item_out_hbm.at[HIST - 2, :, pl.ds(b0, BW)], osem1).wait()
    pltpu.make_async_copy(
        blk0_v, item_out_hbm.at[HIST - 1, :, pl.ds(b0, BW)], osem0).wait()


_gather = functools.partial(
    pl.kernel,
    out_type=(
        jax.ShapeDtypeStruct((D, B), jnp.float32),
        jax.ShapeDtypeStruct((HIST, D, B), jnp.float32),
    ),
    mesh=_MESH,
    scratch_types=[
        pltpu.VMEM((BW,), jnp.int32),
        pltpu.VMEM((IB,), jnp.int32),
        pltpu.VMEM((BW,), jnp.int32),
        pltpu.VMEM((BW,), jnp.int32),
        pltpu.VMEM((BW,), jnp.int32),
        pltpu.VMEM((BW,), jnp.int32),
        pltpu.VMEM((BW, 128), jnp.float32),
        pltpu.VMEM((BW, 128), jnp.float32),
        pltpu.VMEM((D, BW), jnp.float32),
        pltpu.VMEM((D, BW), jnp.float32),
        pltpu.SemaphoreType.DMA,
        pltpu.SemaphoreType.DMA,
        pltpu.SemaphoreType.DMA,
        pltpu.SemaphoreType.DMA,
        pltpu.SemaphoreType.DMA,
    ],
    compiler_params=_PARAMS,
)(_gather_body)


def kernel(user_id, items_ids, user_table, item_table):
    items_flat = items_ids.reshape(B * HIST)
    user_tail = user_table[UCH * 128:].reshape(UREM * D // 128, 128)
    item_tail = item_table[ICH * 128:].reshape(IREM * D // 128, 128)
    user_l, item_l = _retile(user_table.T, item_table.T,
                             user_tail, item_tail)
    uT, iT = _gather(user_id, items_flat, user_l, item_l)
    return uT.T, iT.transpose(2, 0, 1)


# speedup vs baseline: 1.3257x; 1.3257x over previous
import functools

import jax
import jax.numpy as jnp
from jax import lax
from jax.experimental import pallas as pl
from jax.experimental.pallas import tpu as pltpu
from jax.experimental.pallas import tpu_sc as plsc

B = 4096
HIST = 50
D = 32
NC, NS = 2, 16
NW = NC * NS
BW = B // NW
IB = BW * HIST
L = 16
NG = BW // L

UROWS = 100000
IROWS = 1000000
UCH = UROWS // 128
UREM = UROWS - UCH * 128
ICH = IROWS // 128
IREM = IROWS - ICH * 128

_PARAMS = pltpu.CompilerParams(use_tc_tiling_on_sc=True,
                               needs_layout_passes=False)
_MESH = plsc.VectorSubcoreMesh(core_axis_name="c", subcore_axis_name="s",
                               num_cores=NC, num_subcores=NS)


def _transpose_chunk(in_v, tr_v, lanes, n_gl):
    for gl in range(n_gl):
        vals = []
        for h in range(8):
            q = h // 2
            dvec = lanes + (16 * (h % 2))
            vals.append(plsc.load_gather(
                in_v, [dvec, lanes * 0 + (4 * gl + q)]))
        for h in range(8):
            tr_v[gl, pl.ds(16 * h, L)] = vals[h]


def _retile_body(user_tt_hbm, item_tt_hbm, user_tail_hbm, item_tail_hbm,
                 user_l_hbm, item_l_hbm,
                 in0_v, in1_v, tr0_v, tr1_v,
                 isem0, isem1, osem0, osem1):
    wid = lax.axis_index("s") * NC + lax.axis_index("c")
    lanes = lax.iota(jnp.int32, L)
    ins = (in0_v, in1_v)
    trs = (tr0_v, tr1_v)
    isems = (isem0, isem1)
    osems = (osem0, osem1)

    def run(src_hbm, dst_hbm, nfull):
        def fire(t, par):
            c = wid + t * NW
            pltpu.async_copy(src_hbm.at[:, pl.ds(c * 128, 128)],
                             ins[par], isems[par])

        fire(0, 0)

        def step(t, par):
            @pl.when(t + 1 < nfull)
            def _():
                fire(t + 1, 1 - par)
            pltpu.make_async_copy(
                src_hbm.at[:, pl.ds((wid + t * NW) * 128, 128)],
                ins[par], isems[par]).wait()

            @pl.when(t >= 2)
            def _():
                c_old = wid + (t - 2) * NW
                pltpu.make_async_copy(
                    trs[par], dst_hbm.at[pl.ds(c_old * 32, 32)],
                    osems[par]).wait()
            _transpose_chunk(ins[par], trs[par], lanes, 32)
            c = wid + t * NW
            pltpu.async_copy(trs[par], dst_hbm.at[pl.ds(c * 32, 32)],
                             osems[par])

        def pair(p, carry):
            step(p * 2, 0)
            step(p * 2 + 1, 1)
            return carry

        lax.fori_loop(0, nfull // 2, pair, 0)

        def drain(par, t):
            pltpu.make_async_copy(
                trs[par], dst_hbm.at[pl.ds((wid + t * NW) * 32, 32)],
                osems[par]).wait()

        drain(0, nfull - 2)
        drain(1, nfull - 1)

    run(item_tt_hbm, item_l_hbm, ICH // NW)
    run(user_tt_hbm, user_l_hbm, UCH // NW)

    c_extra = (ICH // NW) * NW + wid

    @pl.when(c_extra < ICH)
    def _():
        pltpu.sync_copy(item_tt_hbm.at[:, pl.ds(c_extra * 128, 128)], in0_v)
        _transpose_chunk(in0_v, tr0_v, lanes, 32)
        pltpu.sync_copy(tr0_v, item_l_hbm.at[pl.ds(c_extra * 32, 32)])

    @pl.when(wid == (ICH % NW))
    def _():
        pltpu.sync_copy(item_tail_hbm, tr1_v.at[pl.ds(0, 16)])
        pltpu.sync_copy(tr1_v.at[pl.ds(0, 16)],
                        item_l_hbm.at[pl.ds(IROWS * D // 128 - 16, 16)])

    u_extra = (UCH // NW) * NW + wid

    @pl.when(u_extra < UCH)
    def _():
        pltpu.sync_copy(user_tt_hbm.at[:, pl.ds(u_extra * 128, 128)], in0_v)
        _transpose_chunk(in0_v, tr0_v, lanes, 32)
        pltpu.sync_copy(tr0_v, user_l_hbm.at[pl.ds(u_extra * 32, 32)])

    @pl.when(wid == (UCH % NW))
    def _():
        pltpu.sync_copy(user_tail_hbm, in1_v.at[pl.ds(0, 8)])
        pltpu.sync_copy(in1_v.at[pl.ds(0, 8)],
                        user_l_hbm.at[pl.ds(UROWS * D // 128 - 8, 8)])


_retile = functools.partial(
    pl.kernel,
    out_type=(
        jax.ShapeDtypeStruct((UROWS * D // 128, 128), jnp.float32),
        jax.ShapeDtypeStruct((IROWS * D // 128, 128), jnp.float32),
    ),
    mesh=_MESH,
    scratch_types=[
        pltpu.VMEM((D, 128), jnp.float32),
        pltpu.VMEM((D, 128), jnp.float32),
        pltpu.VMEM((32, 128), jnp.float32),
        pltpu.VMEM((32, 128), jnp.float32),
        pltpu.SemaphoreType.DMA,
        pltpu.SemaphoreType.DMA,
        pltpu.SemaphoreType.DMA,
        pltpu.SemaphoreType.DMA,
    ],
    compiler_params=_PARAMS,
)(_retile_body)


def _extract_transpose(lines_v, col_v, blk_v, lanes):
    for g in range(NG):
        rows = lanes + (g * L)
        cols = col_v[pl.ds(g * L, L)]
        vals = [plsc.load_gather(lines_v, [rows, cols + d])
                for d in range(D)]
        for d in range(D):
            blk_v[d, pl.ds(g * L, L)] = vals[d]


def _gather_body(user_id_hbm, items_hbm, user_l_hbm, item_l_hbm,
                 user_out_hbm, item_out_hbm,
                 uidx_v, iidx_v, lid0_v, lid1_v, col0_v, col1_v,
                 lines0_v, lines1_v, blk0_v, blk1_v,
                 gsem0, gsem1, osem0, osem1, usem):
    wid = lax.axis_index("s") * NC + lax.axis_index("c")
    b0 = wid * BW

    pltpu.sync_copy(user_id_hbm.at[pl.ds(b0, BW)], uidx_v)
    pltpu.sync_copy(items_hbm.at[pl.ds(b0 * HIST, IB)], iidx_v)

    lanes = lax.iota(jnp.int32, L)
    lanes50 = lanes * HIST

    lids = (lid0_v, lid1_v)
    cols = (col0_v, col1_v)
    lines = (lines0_v, lines1_v)
    blks = (blk0_v, blk1_v)
    gsems = (gsem0, gsem1)
    osems = (osem0, osem1)

    def prep_fire(l, par):
        for g in range(NG):
            raw = plsc.load_gather(iidx_v, [lanes50 + (g * L * HIST + l)])
            lids[par][pl.ds(g * L, L)] = lax.shift_right_logical(raw, 2)
            cols[par][pl.ds(g * L, L)] = lax.shift_left(
                lax.bitwise_and(raw, jnp.int32(3)), 5)
        pltpu.async_copy(item_l_hbm.at[lids[par]], lines[par], gsems[par])

    for g in range(NG):
        raw = uidx_v[pl.ds(g * L, L)]
        lid0_v[pl.ds(g * L, L)] = lax.shift_right_logical(raw, 2)
        col0_v[pl.ds(g * L, L)] = lax.shift_left(
            lax.bitwise_and(raw, jnp.int32(3)), 5)
    pltpu.async_copy(user_l_hbm.at[lid0_v], lines0_v, usem)
    pltpu.make_async_copy(user_l_hbm.at[lid0_v], lines0_v, usem).wait()
    _extract_transpose(lines0_v, col0_v, blk0_v, lanes)
    pltpu.sync_copy(blk0_v, user_out_hbm.at[:, pl.ds(b0, BW)])

    prep_fire(0, 1)

    def step(l, par):
        @pl.when(l + 1 < HIST)
        def _():
            prep_fire(l + 1, 1 - par)
        pltpu.make_async_copy(
            item_l_hbm.at[lids[par]], lines[par], gsems[par]).wait()

        @pl.when(l >= 2)
        def _():
            pltpu.make_async_copy(
                blks[par], item_out_hbm.at[l - 2, :, pl.ds(b0, BW)],
                osems[par]).wait()
        _extract_transpose(lines[par], cols[par], blks[par], lanes)
        pltpu.async_copy(blks[par],
                         item_out_hbm.at[l, :, pl.ds(b0, BW)], osems[par])

    def pair(p, carry):
        l = p * 2
        step(l, 1)
        step(l + 1, 0)
        return carry

    lax.fori_loop(0, HIST // 2, pair, 0)

    pltpu.make_async_copy(
        blk1_v, item_out_hbm.at[HIST - 2, :, pl.ds(b0, BW)], osem1).wait()
    pltpu.make_async_copy(
        blk0_v, item_out_hbm.at[HIST - 1, :, pl.ds(b0, BW)], osem0).wait()


_gather = functools.partial(
    pl.kernel,
    out_type=(
        jax.ShapeDtypeStruct((D, B), jnp.float32),
        jax.ShapeDtypeStruct((HIST, D, B), jnp.float32),
    ),
    mesh=_MESH,
    scratch_types=[
        pltpu.VMEM((BW,), jnp.int32),
        pltpu.VMEM((IB,), jnp.int32),
        pltpu.VMEM((BW,), jnp.int32),
        pltpu.VMEM((BW,), jnp.int32),
        pltpu.VMEM((BW,), jnp.int32),
        pltpu.VMEM((BW,), jnp.int32),
        pltpu.VMEM((BW, 128), jnp.float32),
        pltpu.VMEM((BW, 128), jnp.float32),
        pltpu.VMEM((D, BW), jnp.float32),
        pltpu.VMEM((D, BW), jnp.float32),
        pltpu.SemaphoreType.DMA,
        pltpu.SemaphoreType.DMA,
        pltpu.SemaphoreType.DMA,
        pltpu.SemaphoreType.DMA,
        pltpu.SemaphoreType.DMA,
    ],
    compiler_params=_PARAMS,
)(_gather_body)


def kernel(user_id, items_ids, user_table, item_table):
    items_flat = items_ids.reshape(B * HIST)
    user_tail = user_table[UCH * 128:].reshape(UREM * D // 128, 128)
    item_tail = item_table[ICH * 128:].reshape(IREM * D // 128, 128)
    user_l, item_l = _retile(user_table.T, item_table.T,
                             user_tail, item_tail)
    uT, iT = _gather(user_id, items_flat, user_l, item_l)
    return uT.T, iT.transpose(2, 0, 1)


# speedup vs baseline: 1.3514x; 1.0194x over previous
import functools

import jax
import jax.numpy as jnp
from jax import lax
from jax.experimental import pallas as pl
from jax.experimental.pallas import tpu as pltpu
from jax.experimental.pallas import tpu_sc as plsc

B = 4096
HIST = 50
D = 32
NC, NS = 2, 16
NW = NC * NS
BW = B // NW
IB = BW * HIST
L = 16
NG = BW // L

UROWS = 100000
IROWS = 1000000
UCH = UROWS // 128
UREM = UROWS - UCH * 128
ICH = IROWS // 128
IREM = IROWS - ICH * 128

_PARAMS = pltpu.CompilerParams(use_tc_tiling_on_sc=True,
                               needs_layout_passes=False)
_MESH = plsc.VectorSubcoreMesh(core_axis_name="c", subcore_axis_name="s",
                               num_cores=NC, num_subcores=NS)


def _transpose_chunk(in_v, tr_v, lanes, n_gl, dst_idx):
    for h in range(n_gl // 4):
        for d in range(D):
            v = in_v[d, pl.ds(h * L, L)]
            plsc.store_scatter(tr_v, [dst_idx[0] + (h * 4),
                                      dst_idx[1] + d], v)


def _make_dst_idx(lanes):
    r = lax.shift_right_logical(lanes, 2)
    c = lax.shift_left(lax.bitwise_and(lanes, jnp.int32(3)), 5)
    return (r, c)


def _retile_body(user_tt_hbm, item_tt_hbm, user_tail_hbm, item_tail_hbm,
                 user_l_hbm, item_l_hbm,
                 in0_v, in1_v, tr0_v, tr1_v,
                 isem0, isem1, osem0, osem1):
    wid = lax.axis_index("s") * NC + lax.axis_index("c")
    lanes = lax.iota(jnp.int32, L)
    dst_idx = _make_dst_idx(lanes)
    ins = (in0_v, in1_v)
    trs = (tr0_v, tr1_v)
    isems = (isem0, isem1)
    osems = (osem0, osem1)

    def run(src_hbm, dst_hbm, nfull):
        def fire(t, par):
            c = wid + t * NW
            pltpu.async_copy(src_hbm.at[:, pl.ds(c * 128, 128)],
                             ins[par], isems[par])

        fire(0, 0)

        def step(t, par):
            @pl.when(t + 1 < nfull)
            def _():
                fire(t + 1, 1 - par)
            pltpu.make_async_copy(
                src_hbm.at[:, pl.ds((wid + t * NW) * 128, 128)],
                ins[par], isems[par]).wait()

            @pl.when(t >= 2)
            def _():
                c_old = wid + (t - 2) * NW
                pltpu.make_async_copy(
                    trs[par], dst_hbm.at[pl.ds(c_old * 32, 32)],
                    osems[par]).wait()
            _transpose_chunk(ins[par], trs[par], lanes, 32, dst_idx)
            c = wid + t * NW
            pltpu.async_copy(trs[par], dst_hbm.at[pl.ds(c * 32, 32)],
                             osems[par])

        def pair(p, carry):
            step(p * 2, 0)
            step(p * 2 + 1, 1)
            return carry

        lax.fori_loop(0, nfull // 2, pair, 0)

        def drain(par, t):
            pltpu.make_async_copy(
                trs[par], dst_hbm.at[pl.ds((wid + t * NW) * 32, 32)],
                osems[par]).wait()

        drain(0, nfull - 2)
        drain(1, nfull - 1)

    run(item_tt_hbm, item_l_hbm, ICH // NW)
    run(user_tt_hbm, user_l_hbm, UCH // NW)

    c_extra = (ICH // NW) * NW + wid

    @pl.when(c_extra < ICH)
    def _():
        pltpu.sync_copy(item_tt_hbm.at[:, pl.ds(c_extra * 128, 128)], in0_v)
        _transpose_chunk(in0_v, tr0_v, lanes, 32, dst_idx)
        pltpu.sync_copy(tr0_v, item_l_hbm.at[pl.ds(c_extra * 32, 32)])

    @pl.when(wid == (ICH % NW))
    def _():
        pltpu.sync_copy(item_tail_hbm, tr1_v.at[pl.ds(0, 16)])
        pltpu.sync_copy(tr1_v.at[pl.ds(0, 16)],
                        item_l_hbm.at[pl.ds(IROWS * D // 128 - 16, 16)])

    u_extra = (UCH // NW) * NW + wid

    @pl.when(u_extra < UCH)
    def _():
        pltpu.sync_copy(user_tt_hbm.at[:, pl.ds(u_extra * 128, 128)], in0_v)
        _transpose_chunk(in0_v, tr0_v, lanes, 32, dst_idx)
        pltpu.sync_copy(tr0_v, user_l_hbm.at[pl.ds(u_extra * 32, 32)])

    @pl.when(wid == (UCH % NW))
    def _():
        pltpu.sync_copy(user_tail_hbm, in1_v.at[pl.ds(0, 8)])
        pltpu.sync_copy(in1_v.at[pl.ds(0, 8)],
                        user_l_hbm.at[pl.ds(UROWS * D // 128 - 8, 8)])


_retile = functools.partial(
    pl.kernel,
    out_type=(
        jax.ShapeDtypeStruct((UROWS * D // 128, 128), jnp.float32),
        jax.ShapeDtypeStruct((IROWS * D // 128, 128), jnp.float32),
    ),
    mesh=_MESH,
    scratch_types=[
        pltpu.VMEM((D, 128), jnp.float32),
        pltpu.VMEM((D, 128), jnp.float32),
        pltpu.VMEM((32, 128), jnp.float32),
        pltpu.VMEM((32, 128), jnp.float32),
        pltpu.SemaphoreType.DMA,
        pltpu.SemaphoreType.DMA,
        pltpu.SemaphoreType.DMA,
        pltpu.SemaphoreType.DMA,
    ],
    compiler_params=_PARAMS,
)(_retile_body)


def _extract_transpose(lines_v, col_v, blk_v, lanes):
    for g in range(NG):
        rows = lanes + (g * L)
        cols = col_v[pl.ds(g * L, L)]
        vals = [plsc.load_gather(lines_v, [rows, cols + d])
                for d in range(D)]
        for d in range(D):
            blk_v[d, pl.ds(g * L, L)] = vals[d]


def _gather_body(user_id_hbm, items_hbm, user_l_hbm, item_l_hbm,
                 user_out_hbm, item_out_hbm,
                 uidx_v, iidx_v, lid0_v, lid1_v, col0_v, col1_v,
                 lines0_v, lines1_v, blk0_v, blk1_v,
                 gsem0, gsem1, osem0, osem1, usem):
    wid = lax.axis_index("s") * NC + lax.axis_index("c")
    b0 = wid * BW

    pltpu.sync_copy(user_id_hbm.at[pl.ds(b0, BW)], uidx_v)
    pltpu.sync_copy(items_hbm.at[pl.ds(b0 * HIST, IB)], iidx_v)

    lanes = lax.iota(jnp.int32, L)
    lanes50 = lanes * HIST

    lids = (lid0_v, lid1_v)
    cols = (col0_v, col1_v)
    lines = (lines0_v, lines1_v)
    blks = (blk0_v, blk1_v)
    gsems = (gsem0, gsem1)
    osems = (osem0, osem1)

    def prep_fire(l, par):
        for g in range(NG):
            raw = plsc.load_gather(iidx_v, [lanes50 + (g * L * HIST + l)])
            lids[par][pl.ds(g * L, L)] = lax.shift_right_logical(raw, 2)
            cols[par][pl.ds(g * L, L)] = lax.shift_left(
                lax.bitwise_and(raw, jnp.int32(3)), 5)
        pltpu.async_copy(item_l_hbm.at[lids[par]], lines[par], gsems[par])

    for g in range(NG):
        raw = uidx_v[pl.ds(g * L, L)]
        lid0_v[pl.ds(g * L, L)] = lax.shift_right_logical(raw, 2)
        col0_v[pl.ds(g * L, L)] = lax.shift_left(
            lax.bitwise_and(raw, jnp.int32(3)), 5)
    pltpu.async_copy(user_l_hbm.at[lid0_v], lines0_v, usem)
    pltpu.make_async_copy(user_l_hbm.at[lid0_v], lines0_v, usem).wait()
    _extract_transpose(lines0_v, col0_v, blk0_v, lanes)
    pltpu.sync_copy(blk0_v, user_out_hbm.at[:, pl.ds(b0, BW)])

    prep_fire(0, 1)

    def step(l, par):
        @pl.when(l + 1 < HIST)
        def _():
            prep_fire(l + 1, 1 - par)
        pltpu.make_async_copy(
            item_l_hbm.at[lids[par]], lines[par], gsems[par]).wait()

        @pl.when(l >= 2)
        def _():
            pltpu.make_async_copy(
                blks[par], item_out_hbm.at[l - 2, :, pl.ds(b0, BW)],
                osems[par]).wait()
        _extract_transpose(lines[par], cols[par], blks[par], lanes)
        pltpu.async_copy(blks[par],
                         item_out_hbm.at[l, :, pl.ds(b0, BW)], osems[par])

    def pair(p, carry):
        l = p * 2
        step(l, 1)
        step(l + 1, 0)
        return carry

    lax.fori_loop(0, HIST // 2, pair, 0)

    pltpu.make_async_copy(
        blk1_v, item_out_hbm.at[HIST - 2, :, pl.ds(b0, BW)], osem1).wait()
    pltpu.make_async_copy(
        blk0_v, item_out_hbm.at[HIST - 1, :, pl.ds(b0, BW)], osem0).wait()


_gather = functools.partial(
    pl.kernel,
    out_type=(
        jax.ShapeDtypeStruct((D, B), jnp.float32),
        jax.ShapeDtypeStruct((HIST, D, B), jnp.float32),
    ),
    mesh=_MESH,
    scratch_types=[
        pltpu.VMEM((BW,), jnp.int32),
        pltpu.VMEM((IB,), jnp.int32),
        pltpu.VMEM((BW,), jnp.int32),
        pltpu.VMEM((BW,), jnp.int32),
        pltpu.VMEM((BW,), jnp.int32),
        pltpu.VMEM((BW,), jnp.int32),
        pltpu.VMEM((BW, 128), jnp.float32),
        pltpu.VMEM((BW, 128), jnp.float32),
        pltpu.VMEM((D, BW), jnp.float32),
        pltpu.VMEM((D, BW), jnp.float32),
        pltpu.SemaphoreType.DMA,
        pltpu.SemaphoreType.DMA,
        pltpu.SemaphoreType.DMA,
        pltpu.SemaphoreType.DMA,
        pltpu.SemaphoreType.DMA,
    ],
    compiler_params=_PARAMS,
)(_gather_body)


def kernel(user_id, items_ids, user_table, item_table):
    items_flat = items_ids.reshape(B * HIST)
    user_tail = user_table[UCH * 128:].reshape(UREM * D // 128, 128)
    item_tail = item_table[ICH * 128:].reshape(IREM * D // 128, 128)
    user_l, item_l = _retile(user_table.T, item_table.T,
                             user_tail, item_tail)
    uT, iT = _gather(user_id, items_flat, user_l, item_l)
    return uT.T, iT.transpose(2, 0, 1)


# speedup vs baseline: 1.6646x; 1.2317x over previous
import functools

import jax
import jax.numpy as jnp
from jax import lax
from jax.experimental import pallas as pl
from jax.experimental.pallas import tpu as pltpu
from jax.experimental.pallas import tpu_sc as plsc

B = 4096
HIST = 50
D = 32
NC, NS = 2, 16
NW = NC * NS
BW = B // NW
IB = BW * HIST
L = 16
NG = BW // L
RT = D // 8


def _transpose_rows(rows_v, blk_v, lanes):
    for g in range(NG):
        rows = lanes + (g * L)
        vals = [plsc.load_gather(rows_v, [rows, lanes * 0 + d])
                for d in range(D)]
        for d in range(D):
            blk_v[d, pl.ds(g * L, L)] = vals[d]


def _gather_body(user_id_hbm, items_hbm, user_rows_hbm, item_rows_hbm,
                 user_out_hbm, item_out_hbm,
                 uidx_v, iidx_v, lid0_v, lid1_v,
                 rows0_v, rows1_v, blk0_v, blk1_v,
                 gsem0, gsem1, osem0, osem1, usem):
    wid = lax.axis_index("s") * NC + lax.axis_index("c")
    b0 = wid * BW

    pltpu.sync_copy(user_id_hbm.at[pl.ds(b0, BW)], uidx_v)
    pltpu.sync_copy(items_hbm.at[pl.ds(b0 * HIST, IB)], iidx_v)

    lanes = lax.iota(jnp.int32, L)
    lanes50 = lanes * HIST

    lids = (lid0_v, lid1_v)
    rows = (rows0_v, rows1_v)
    blks = (blk0_v, blk1_v)
    gsems = (gsem0, gsem1)
    osems = (osem0, osem1)

    def prep_fire(l, par):
        for g in range(NG):
            lids[par][pl.ds(g * L, L)] = plsc.load_gather(
                iidx_v, [lanes50 + (g * L * HIST + l)])
        pltpu.async_copy(item_rows_hbm.at[lids[par]], rows[par], gsems[par])

    pltpu.async_copy(user_rows_hbm.at[uidx_v], rows0_v, usem)
    prep_fire(0, 1)
    pltpu.make_async_copy(user_rows_hbm.at[uidx_v], rows0_v, usem).wait()
    _transpose_rows(rows0_v, blk0_v, lanes)
    for r in range(RT):
        pltpu.sync_copy(blk0_v.at[pl.ds(8 * r, 8)], user_out_hbm.at[r, wid])

    def step(l, par):
        @pl.when(l + 1 < HIST)
        def _():
            prep_fire(l + 1, 1 - par)
        pltpu.make_async_copy(
            item_rows_hbm.at[lids[par]], rows[par], gsems[par]).wait()

        @pl.when(l >= 2)
        def _():
            for r in range(RT):
                pltpu.make_async_copy(
                    blks[par].at[pl.ds(8 * r, 8)],
                    item_out_hbm.at[l - 2, r, wid], osems[par]).wait()
        _transpose_rows(rows[par], blks[par], lanes)
        for r in range(RT):
            pltpu.async_copy(blks[par].at[pl.ds(8 * r, 8)],
                             item_out_hbm.at[l, r, wid], osems[par])

    def pair(p, carry):
        l = p * 2
        step(l, 1)
        step(l + 1, 0)
        return carry

    lax.fori_loop(0, HIST // 2, pair, 0)

    for r in range(RT):
        pltpu.make_async_copy(
            blk1_v.at[pl.ds(8 * r, 8)],
            item_out_hbm.at[HIST - 2, r, wid], osem1).wait()
    for r in range(RT):
        pltpu.make_async_copy(
            blk0_v.at[pl.ds(8 * r, 8)],
            item_out_hbm.at[HIST - 1, r, wid], osem0).wait()


_gather = functools.partial(
    pl.kernel,
    out_type=(
        jax.ShapeDtypeStruct((RT, NW, 8, BW), jnp.float32),
        jax.ShapeDtypeStruct((HIST, RT, NW, 8, BW), jnp.float32),
    ),
    mesh=plsc.VectorSubcoreMesh(core_axis_name="c", subcore_axis_name="s",
                                num_cores=NC, num_subcores=NS),
    scratch_types=[
        pltpu.VMEM((BW,), jnp.int32),
        pltpu.VMEM((IB,), jnp.int32),
        pltpu.VMEM((BW,), jnp.int32),
        pltpu.VMEM((BW,), jnp.int32),
        pltpu.VMEM((BW, D), jnp.float32),
        pltpu.VMEM((BW, D), jnp.float32),
        pltpu.VMEM((D, BW), jnp.float32),
        pltpu.VMEM((D, BW), jnp.float32),
        pltpu.SemaphoreType.DMA,
        pltpu.SemaphoreType.DMA,
        pltpu.SemaphoreType.DMA,
        pltpu.SemaphoreType.DMA,
        pltpu.SemaphoreType.DMA,
    ],
    compiler_params=pltpu.CompilerParams(use_tc_tiling_on_sc=False,
                                         needs_layout_passes=False),
)(_gather_body)


def kernel(user_id, items_ids, user_table, item_table):
    items_flat = items_ids.reshape(B * HIST)
    u4, i5 = _gather(user_id, items_flat, user_table, item_table)
    user_out = u4.transpose(1, 3, 0, 2).reshape(B, D)
    item_out = i5.transpose(2, 4, 0, 1, 3).reshape(B, HIST, D)
    return user_out, item_out
